# baseline (speedup 1.0000x reference)
import functools

import jax
import jax.numpy as jnp
from jax import lax
from jax.experimental import pallas as pl
from jax.experimental.pallas import tpu as pltpu
from jax.experimental.pallas import tpu_sc as plsc

N = 10000
E = 320000
D = 128
NP = 10240

NC = 2
NS = 16
NW = NC * NS
EPW = E // NW
K = 80
NCHUNK = EPW // K
NPAIR = (NCHUNK - 1) // 2
RPT = NP // NS


def _sc_aggregate(src_hbm, dst_hbm, nodes_hbm, ef_hbm, s_out, deg_out,
                  sidx_a, didx_a, rows_na, rows_ea,
                  sidx_b, didx_b, rows_nb, rows_eb,
                  ones_v, acc_sh, deg_sh,
                  gs_a, es_a, ss_a, gs_b, es_b, ss_b):
    c = lax.axis_index("c")
    s = lax.axis_index("s")
    wid = s * NC + c

    zero16 = jnp.zeros((16,), jnp.float32)

    def zero_rows(r, _):
        def zero_cols(j, _):
            rows_na[r, pl.ds(j * 16, 16)] = zero16
            return 0
        return lax.fori_loop(0, D // 16, zero_cols, 0)

    lax.fori_loop(0, K, zero_rows, 0)

    def zero_ones(i, _):
        ones_v[pl.ds(i * 16, 16)] = zero16
        return 0
    lax.fori_loop(0, K // 16, zero_ones, 0)

    rbase = s * RPT
    for t in range(RPT // K):
        pltpu.sync_copy(rows_na.at[pl.ds(0, K)],
                        acc_sh.at[pl.ds(rbase + t * K, K)])
        pltpu.sync_copy(ones_v.at[pl.ds(0, K)],
                        deg_sh.at[pl.ds(rbase + t * K, K)])

    one16 = jnp.ones((16,), jnp.float32)

    def fill_ones(i, _):
        ones_v[pl.ds(i * 16, 16)] = one16
        return 0
    lax.fori_loop(0, K // 16, fill_ones, 0)

    plsc.subcore_barrier()

    ebase = wid * EPW

    def issue_loads(off, sidx, didx, rows_n, rows_e, gsem, esem):
        pltpu.sync_copy(src_hbm.at[pl.ds(off, K)], sidx)
        pltpu.sync_copy(dst_hbm.at[pl.ds(off, K)], didx)
        gd = pltpu.async_copy(nodes_hbm.at[sidx], rows_n, gsem)
        ed = pltpu.async_copy(ef_hbm.at[pl.ds(off, K)], rows_e, esem)
        return gd, ed

    def add_rows(rows_n, rows_e):
        def addrow(r, _):
            for j in range(D // 16):
                sl = pl.ds(j * 16, 16)
                plsc.addupdate(rows_n.at[r, sl], rows_e[r, sl])
            return 0
        lax.fori_loop(0, K, addrow, 0)

    def issue_scatters(didx, rows_n, rows_e, ssem):
        pltpu.async_copy(rows_n, acc_sh.at[didx], ssem, add=True)
        pltpu.async_copy(ones_v, deg_sh.at[didx], ssem, add=True)

    def wait_scatters(didx, rows_n, rows_e, ssem):
        pltpu.make_async_copy(rows_n, acc_sh.at[didx], ssem).wait()
        pltpu.make_async_copy(ones_v, deg_sh.at[didx], ssem).wait()

    def pair(i, _):
        off_a = ebase + (2 * i) * K
        off_b = off_a + K
        gd_a, ed_a = issue_loads(off_a, sidx_a, didx_a, rows_na, rows_ea,
                                 gs_a, es_a)

        @pl.when(i > 0)
        def _():
            wait_scatters(didx_b, rows_nb, rows_eb, ss_b)

        gd_b, ed_b = issue_loads(off_b, sidx_b, didx_b, rows_nb, rows_eb,
                                 gs_b, es_b)
        gd_a.wait()
        ed_a.wait()
        add_rows(rows_na, rows_ea)
        issue_scatters(didx_a, rows_na, rows_ea, ss_a)
        gd_b.wait()
        ed_b.wait()
        add_rows(rows_nb, rows_eb)
        wait_scatters(didx_a, rows_na, rows_ea, ss_a)
        issue_scatters(didx_b, rows_nb, rows_eb, ss_b)
        return 0

    lax.fori_loop(0, NPAIR, pair, 0)

    off = ebase + (NCHUNK - 1) * K
    gd_a, ed_a = issue_loads(off, sidx_a, didx_a, rows_na, rows_ea,
                             gs_a, es_a)
    wait_scatters(didx_b, rows_nb, rows_eb, ss_b)
    gd_a.wait()
    ed_a.wait()
    add_rows(rows_na, rows_ea)
    issue_scatters(didx_a, rows_na, rows_ea, ss_a)
    wait_scatters(didx_a, rows_na, rows_ea, ss_a)

    plsc.subcore_barrier()

    pltpu.sync_copy(acc_sh.at[pl.ds(rbase, RPT)],
                    s_out.at[c, pl.ds(rbase, RPT)])
    pltpu.sync_copy(deg_sh.at[pl.ds(rbase, RPT)],
                    deg_out.at[pl.ds(c * NP + rbase, RPT)])


_sc_call = functools.partial(
    pl.kernel,
    out_type=[
        jax.ShapeDtypeStruct((NC, NP, D), jnp.float32),
        jax.ShapeDtypeStruct((NC * NP,), jnp.float32),
    ],
    mesh=plsc.VectorSubcoreMesh(core_axis_name="c", subcore_axis_name="s"),
    scratch_types=[
        pltpu.VMEM((K,), jnp.int32),
        pltpu.VMEM((K,), jnp.int32),
        pltpu.VMEM((K, D), jnp.float32),
        pltpu.VMEM((K, D), jnp.float32),
        pltpu.VMEM((K,), jnp.int32),
        pltpu.VMEM((K,), jnp.int32),
        pltpu.VMEM((K, D), jnp.float32),
        pltpu.VMEM((K, D), jnp.float32),
        pltpu.VMEM((K,), jnp.float32),
        pltpu.VMEM_SHARED((NP, D), jnp.float32),
        pltpu.VMEM_SHARED((NP,), jnp.float32),
        pltpu.SemaphoreType.DMA,
        pltpu.SemaphoreType.DMA,
        pltpu.SemaphoreType.DMA,
        pltpu.SemaphoreType.DMA,
        pltpu.SemaphoreType.DMA,
        pltpu.SemaphoreType.DMA,
    ],
)(_sc_aggregate)


def _tc_mlp(x_ref, sp_ref, dp_ref, eps_ref, w1_ref, b1_ref, g1_ref, bt1_ref,
            w2_ref, b2_ref, g2_ref, bt2_ref, o_ref):
    s = sp_ref[0, :N] + sp_ref[1, :N]
    deg = dp_ref[:N] + dp_ref[NP:NP + N]
    h = ((1.0 + eps_ref[0, 0]) * x_ref[...]
         + s / jnp.maximum(deg[:, None], 1.0))

    h = jnp.dot(h, w1_ref[...], preferred_element_type=jnp.float32) + b1_ref[...]
    mean = jnp.mean(h, axis=0, keepdims=True)
    var = jnp.mean((h - mean) ** 2, axis=0, keepdims=True)
    h = g1_ref[...] * (h - mean) * lax.rsqrt(var + 1e-5) + bt1_ref[...]
    h = jnp.maximum(h, 0.0)

    h = jnp.dot(h, w2_ref[...], preferred_element_type=jnp.float32) + b2_ref[...]
    mean = jnp.mean(h, axis=0, keepdims=True)
    var = jnp.mean((h - mean) ** 2, axis=0, keepdims=True)
    h = g2_ref[...] * (h - mean) * lax.rsqrt(var + 1e-5) + bt2_ref[...]
    o_ref[...] = jnp.maximum(h, 0.0)


def kernel(node_feats, edge_index, edge_feats, eps, W1, b1, g1, beta1,
           W2, b2, g2, beta2):
    src = edge_index[0]
    dst = edge_index[1]

    s_part, deg_part = _sc_call(src, dst, node_feats, edge_feats)

    out = pl.pallas_call(
        _tc_mlp,
        out_shape=jax.ShapeDtypeStruct((N, D), jnp.float32),
    )(
        node_feats, s_part, deg_part,
        eps.reshape(1, 1),
        W1, b1.reshape(1, -1), g1.reshape(1, -1), beta1.reshape(1, -1),
        W2, b2.reshape(1, -1), g2.reshape(1, -1), beta2.reshape(1, -1),
    )
    return out


# speedup vs baseline: 1.1244x; 1.1244x over previous
import functools

import jax
import jax.numpy as jnp
from jax import lax
from jax.experimental import pallas as pl
from jax.experimental.pallas import tpu as pltpu
from jax.experimental.pallas import tpu_sc as plsc

N = 10000
E = 320000
D = 128
ACCR = 10120
DEGP = 10240

NC = 2
NS = 16
NW = NC * NS
EPW = E // NW
K = 80
NCHUNK = EPW // K
NBLK = 5
CB = NCHUNK // NBLK
NPAIRB = (CB - 1) // 2
RPT = 632


def _sc_aggregate(src_hbm, dst_hbm, nodes_hbm, ef_hbm, s_out, deg_out,
                  sidx_blk, didx_blk, rows_na, rows_ea, rows_nb, rows_eb,
                  ones_v, acc_sh, deg_sh,
                  gs_a, es_a, ss_a, gs_b, es_b, ss_b, dg_s):
    c = lax.axis_index("c")
    s = lax.axis_index("s")
    wid = s * NC + c

    zero16 = jnp.zeros((16,), jnp.float32)

    def zero_rows(r, _):
        def zero_cols(j, _):
            rows_na[r, pl.ds(j * 16, 16)] = zero16
            return 0
        return lax.fori_loop(0, D // 16, zero_cols, 0)

    lax.fori_loop(0, K, zero_rows, 0)

    def zero_ones(i, _):
        ones_v[pl.ds(i * 16, 16)] = zero16
        return 0
    lax.fori_loop(0, K // 16, zero_ones, 0)

    rbase = s * RPT
    dbase = s * (DEGP // NS)
    for t in range(640 // K):
        pltpu.sync_copy(rows_na.at[pl.ds(0, K)],
                        acc_sh.at[pl.ds(rbase + t * K, K)])
        pltpu.sync_copy(ones_v.at[pl.ds(0, K)],
                        deg_sh.at[pl.ds(dbase + t * K, K)])

    one16 = jnp.ones((16,), jnp.float32)

    def fill_ones(i, _):
        ones_v[pl.ds(i * 16, 16)] = one16
        return 0
    lax.fori_loop(0, K // 16, fill_ones, 0)

    plsc.subcore_barrier()

    def issue_loads(blk, j, rows_n, rows_e, gsem, esem):
        gd = pltpu.async_copy(nodes_hbm.at[sidx_blk.at[j]], rows_n, gsem)
        off = ((wid * NCHUNK + blk * CB + j) * K)
        ed = pltpu.async_copy(ef_hbm.at[pl.ds(off, K)], rows_e, esem)
        return gd, ed

    def issue_scatters(j, rows_n, rows_e, ssem):
        pltpu.async_copy(rows_n, acc_sh.at[didx_blk.at[j]], ssem, add=True)
        pltpu.async_copy(rows_e, acc_sh.at[didx_blk.at[j]], ssem, add=True)
        pltpu.async_copy(ones_v, deg_sh.at[didx_blk.at[j]], dg_s, add=True)

    def wait_scatters(j, rows_n, rows_e, ssem):
        pltpu.make_async_copy(rows_n, acc_sh.at[didx_blk.at[j]], ssem).wait()
        pltpu.make_async_copy(rows_e, acc_sh.at[didx_blk.at[j]], ssem).wait()

    def block(blk, _):
        pltpu.sync_copy(src_hbm.at[wid, blk], sidx_blk)
        pltpu.sync_copy(dst_hbm.at[wid, blk], didx_blk)

        def pair(i, _):
            ja = 2 * i
            jb = 2 * i + 1
            gd_a, ed_a = issue_loads(blk, ja, rows_na, rows_ea, gs_a, es_a)

            @pl.when(i > 0)
            def _():
                wait_scatters(jb - 2, rows_nb, rows_eb, ss_b)

            gd_b, ed_b = issue_loads(blk, jb, rows_nb, rows_eb, gs_b, es_b)
            gd_a.wait()
            ed_a.wait()
            issue_scatters(ja, rows_na, rows_ea, ss_a)
            gd_b.wait()
            ed_b.wait()
            issue_scatters(jb, rows_nb, rows_eb, ss_b)
            wait_scatters(ja, rows_na, rows_ea, ss_a)
            return 0

        lax.fori_loop(0, NPAIRB, pair, 0)

        jl = CB - 1
        gd_a, ed_a = issue_loads(blk, jl, rows_na, rows_ea, gs_a, es_a)
        wait_scatters(jl - 2, rows_nb, rows_eb, ss_b)
        gd_a.wait()
        ed_a.wait()
        issue_scatters(jl, rows_na, rows_ea, ss_a)
        wait_scatters(jl, rows_na, rows_ea, ss_a)

        def drain_deg(j, _):
            pltpu.make_async_copy(ones_v, deg_sh.at[didx_blk.at[j]],
                                  dg_s).wait()
            return 0
        lax.fori_loop(0, CB, drain_deg, 0)
        return 0

    lax.fori_loop(0, NBLK, block, 0)

    plsc.subcore_barrier()

    pltpu.sync_copy(deg_sh.at[pl.ds(dbase, DEGP // NS)],
                    deg_out.at[pl.ds(c * DEGP + dbase, DEGP // NS)])

    @pl.when(s < NS - 1)
    def _():
        pltpu.sync_copy(acc_sh.at[pl.ds(rbase, RPT)],
                        s_out.at[c, pl.ds(rbase, RPT)])

    @pl.when(s == NS - 1)
    def _():
        pltpu.sync_copy(acc_sh.at[pl.ds((NS - 1) * RPT, N - (NS - 1) * RPT)],
                        s_out.at[c, pl.ds((NS - 1) * RPT, N - (NS - 1) * RPT)])


_sc_call = functools.partial(
    pl.kernel,
    out_type=[
        jax.ShapeDtypeStruct((NC, N, D), jnp.float32),
        jax.ShapeDtypeStruct((NC * DEGP,), jnp.float32),
    ],
    mesh=plsc.VectorSubcoreMesh(core_axis_name="c", subcore_axis_name="s"),
    scratch_types=[
        pltpu.VMEM((CB, K), jnp.int32),
        pltpu.VMEM((CB, K), jnp.int32),
        pltpu.VMEM((K, D), jnp.float32),
        pltpu.VMEM((K, D), jnp.float32),
        pltpu.VMEM((K, D), jnp.float32),
        pltpu.VMEM((K, D), jnp.float32),
        pltpu.VMEM((K,), jnp.float32),
        pltpu.VMEM_SHARED((ACCR, D), jnp.float32),
        pltpu.VMEM_SHARED((DEGP,), jnp.float32),
        pltpu.SemaphoreType.DMA,
        pltpu.SemaphoreType.DMA,
        pltpu.SemaphoreType.DMA,
        pltpu.SemaphoreType.DMA,
        pltpu.SemaphoreType.DMA,
        pltpu.SemaphoreType.DMA,
        pltpu.SemaphoreType.DMA,
    ],
)(_sc_aggregate)


def _tc_mlp(x_ref, sp_ref, dp_ref, eps_ref, w1_ref, b1_ref, g1_ref, bt1_ref,
            w2_ref, b2_ref, g2_ref, bt2_ref, o_ref):
    s = sp_ref[0] + sp_ref[1]
    deg = dp_ref[:N] + dp_ref[DEGP:DEGP + N]
    h = ((1.0 + eps_ref[0, 0]) * x_ref[...]
         + s / jnp.maximum(deg[:, None], 1.0))

    h = jnp.dot(h, w1_ref[...], preferred_element_type=jnp.float32) + b1_ref[...]
    mean = jnp.mean(h, axis=0, keepdims=True)
    var = jnp.mean((h - mean) ** 2, axis=0, keepdims=True)
    h = g1_ref[...] * (h - mean) * lax.rsqrt(var + 1e-5) + bt1_ref[...]
    h = jnp.maximum(h, 0.0)

    h = jnp.dot(h, w2_ref[...], preferred_element_type=jnp.float32) + b2_ref[...]
    mean = jnp.mean(h, axis=0, keepdims=True)
    var = jnp.mean((h - mean) ** 2, axis=0, keepdims=True)
    h = g2_ref[...] * (h - mean) * lax.rsqrt(var + 1e-5) + bt2_ref[...]
    o_ref[...] = jnp.maximum(h, 0.0)


def kernel(node_feats, edge_index, edge_feats, eps, W1, b1, g1, beta1,
           W2, b2, g2, beta2):
    src = edge_index[0].reshape(NW, NBLK, CB, K)
    dst = edge_index[1].reshape(NW, NBLK, CB, K)

    s_part, deg_part = _sc_call(src, dst, node_feats, edge_feats)

    out = pl.pallas_call(
        _tc_mlp,
        out_shape=jax.ShapeDtypeStruct((N, D), jnp.float32),
    )(
        node_feats, s_part, deg_part,
        eps.reshape(1, 1),
        W1, b1.reshape(1, -1), g1.reshape(1, -1), beta1.reshape(1, -1),
        W2, b2.reshape(1, -1), g2.reshape(1, -1), beta2.reshape(1, -1),
    )
    return out


# speedup vs baseline: 1.1345x; 1.0089x over previous
import functools

import jax
import jax.numpy as jnp
from jax import lax
from jax.experimental import pallas as pl
from jax.experimental.pallas import tpu as pltpu
from jax.experimental.pallas import tpu_sc as plsc

N = 10000
E = 320000
D = 128
ACCR = 10120
DEGP = 10240

NC = 2
NS = 16
NW = NC * NS
EPW = E // NW
K = 80
NCHUNK = EPW // K
NBLK = 5
CB = NCHUNK // NBLK
NPAIRB = (CB - 1) // 2
RPT = 632


def _sc_aggregate(src_hbm, dst_hbm, nodes_hbm, ef_hbm, s_out, deg_out,
                  sidx_blk, didx_blk, rows_na, rows_ea, rows_nb, rows_eb,
                  ones_v, acc_sh, deg_sh,
                  gs_a, es_a, ss_a, gs_b, es_b, ss_b, dg_s):
    c = lax.axis_index("c")
    s = lax.axis_index("s")
    wid = s * NC + c

    zero16 = jnp.zeros((16,), jnp.float32)

    def zero_rows(r, _):
        def zero_cols(j, _):
            rows_na[r, pl.ds(j * 16, 16)] = zero16
            return 0
        return lax.fori_loop(0, D // 16, zero_cols, 0)

    lax.fori_loop(0, K, zero_rows, 0)

    def zero_ones(i, _):
        ones_v[pl.ds(i * 16, 16)] = zero16
        return 0
    lax.fori_loop(0, K // 16, zero_ones, 0)

    rbase = s * RPT
    dbase = s * (DEGP // NS)
    for t in range(640 // K):
        pltpu.sync_copy(rows_na.at[pl.ds(0, K)],
                        acc_sh.at[pl.ds(rbase + t * K, K)])
        pltpu.sync_copy(ones_v.at[pl.ds(0, K)],
                        deg_sh.at[pl.ds(dbase + t * K, K)])

    one16 = jnp.ones((16,), jnp.float32)

    def fill_ones(i, _):
        ones_v[pl.ds(i * 16, 16)] = one16
        return 0
    lax.fori_loop(0, K // 16, fill_ones, 0)

    plsc.subcore_barrier()

    def issue_loads(blk, j, rows_n, rows_e, gsem, esem):
        gd = pltpu.async_copy(nodes_hbm.at[sidx_blk.at[j]], rows_n, gsem)
        off = ((wid * NCHUNK + blk * CB + j) * K)
        ed = pltpu.async_copy(ef_hbm.at[pl.ds(off, K)], rows_e, esem)
        return gd, ed

    def issue_scatters(j, rows_n, rows_e, ssem):
        pltpu.async_copy(rows_n, acc_sh.at[didx_blk.at[j]], ssem, add=True)
        pltpu.async_copy(rows_e, acc_sh.at[didx_blk.at[j]], ssem, add=True)
        pltpu.async_copy(ones_v, deg_sh.at[didx_blk.at[j]], dg_s, add=True)

    def wait_scatters(j, rows_n, rows_e, ssem):
        pltpu.make_async_copy(rows_n, acc_sh.at[didx_blk.at[j]], ssem).wait()
        pltpu.make_async_copy(rows_e, acc_sh.at[didx_blk.at[j]], ssem).wait()

    def block(blk, _):
        pltpu.sync_copy(src_hbm.at[wid, blk], sidx_blk)
        pltpu.sync_copy(dst_hbm.at[wid, blk], didx_blk)

        def pair(i, _):
            ja = 2 * i
            jb = 2 * i + 1
            gd_a, ed_a = issue_loads(blk, ja, rows_na, rows_ea, gs_a, es_a)

            @pl.when(i > 0)
            def _():
                wait_scatters(jb - 2, rows_nb, rows_eb, ss_b)

            gd_b, ed_b = issue_loads(blk, jb, rows_nb, rows_eb, gs_b, es_b)
            gd_a.wait()
            ed_a.wait()
            issue_scatters(ja, rows_na, rows_ea, ss_a)
            gd_b.wait()
            ed_b.wait()
            issue_scatters(jb, rows_nb, rows_eb, ss_b)
            wait_scatters(ja, rows_na, rows_ea, ss_a)
            return 0

        lax.fori_loop(0, NPAIRB, pair, 0)

        jl = CB - 1
        gd_a, ed_a = issue_loads(blk, jl, rows_na, rows_ea, gs_a, es_a)
        wait_scatters(jl - 2, rows_nb, rows_eb, ss_b)
        gd_a.wait()
        ed_a.wait()
        issue_scatters(jl, rows_na, rows_ea, ss_a)
        wait_scatters(jl, rows_na, rows_ea, ss_a)

        def drain_deg(j, _):
            pltpu.make_async_copy(ones_v, deg_sh.at[didx_blk.at[j]],
                                  dg_s).wait()
            return 0
        lax.fori_loop(0, CB, drain_deg, 0)
        return 0

    lax.fori_loop(0, NBLK, block, 0)

    plsc.subcore_barrier()

    pltpu.sync_copy(deg_sh.at[pl.ds(dbase, DEGP // NS)],
                    deg_out.at[pl.ds(c * DEGP + dbase, DEGP // NS)])

    @pl.when(s < NS - 1)
    def _():
        pltpu.sync_copy(acc_sh.at[pl.ds(rbase, RPT)],
                        s_out.at[c, pl.ds(rbase, RPT)])

    @pl.when(s == NS - 1)
    def _():
        pltpu.sync_copy(acc_sh.at[pl.ds((NS - 1) * RPT, N - (NS - 1) * RPT)],
                        s_out.at[c, pl.ds((NS - 1) * RPT, N - (NS - 1) * RPT)])


_sc_call = functools.partial(
    pl.kernel,
    out_type=[
        jax.ShapeDtypeStruct((NC, N, D), jnp.float32),
        jax.ShapeDtypeStruct((NC * DEGP,), jnp.float32),
    ],
    mesh=plsc.VectorSubcoreMesh(core_axis_name="c", subcore_axis_name="s"),
    scratch_types=[
        pltpu.VMEM((CB, K), jnp.int32),
        pltpu.VMEM((CB, K), jnp.int32),
        pltpu.VMEM((K, D), jnp.float32),
        pltpu.VMEM((K, D), jnp.float32),
        pltpu.VMEM((K, D), jnp.float32),
        pltpu.VMEM((K, D), jnp.float32),
        pltpu.VMEM((K,), jnp.float32),
        pltpu.VMEM_SHARED((ACCR, D), jnp.float32),
        pltpu.VMEM_SHARED((DEGP,), jnp.float32),
        pltpu.SemaphoreType.DMA,
        pltpu.SemaphoreType.DMA,
        pltpu.SemaphoreType.DMA,
        pltpu.SemaphoreType.DMA,
        pltpu.SemaphoreType.DMA,
        pltpu.SemaphoreType.DMA,
        pltpu.SemaphoreType.DMA,
    ],
)(_sc_aggregate)


def _bn_relu(h, g, beta):
    mean = jnp.mean(h, axis=0, keepdims=True)
    var = jnp.mean(h * h, axis=0, keepdims=True) - mean * mean
    return jnp.maximum(g * (h - mean) * lax.rsqrt(var + 1e-5) + beta, 0.0)


def _tc_mlp(x_ref, sp_ref, dp_ref, eps_ref, w1_ref, b1_ref, g1_ref, bt1_ref,
            w2_ref, b2_ref, g2_ref, bt2_ref, o_ref):
    s = sp_ref[0] + sp_ref[1]
    deg = dp_ref[:N] + dp_ref[DEGP:DEGP + N]
    h = ((1.0 + eps_ref[0, 0]) * x_ref[...]
         + s / jnp.maximum(deg[:, None], 1.0))

    h = jnp.dot(h, w1_ref[...], preferred_element_type=jnp.float32) + b1_ref[...]
    h = _bn_relu(h, g1_ref[...], bt1_ref[...])
    h = jnp.dot(h, w2_ref[...], preferred_element_type=jnp.float32) + b2_ref[...]
    o_ref[...] = _bn_relu(h, g2_ref[...], bt2_ref[...])


def kernel(node_feats, edge_index, edge_feats, eps, W1, b1, g1, beta1,
           W2, b2, g2, beta2):
    src = edge_index[0].reshape(NW, NBLK, CB, K)
    dst = edge_index[1].reshape(NW, NBLK, CB, K)

    s_part, deg_part = _sc_call(src, dst, node_feats, edge_feats)

    out = pl.pallas_call(
        _tc_mlp,
        out_shape=jax.ShapeDtypeStruct((N, D), jnp.float32),
    )(
        node_feats, s_part, deg_part,
        eps.reshape(1, 1),
        W1, b1.reshape(1, -1), g1.reshape(1, -1), beta1.reshape(1, -1),
        W2, b2.reshape(1, -1), g2.reshape(1, -1), beta2.reshape(1, -1),
    )
    return out


# speedup vs baseline: 1.1391x; 1.0041x over previous
import functools

import jax
import jax.numpy as jnp
from jax import lax
from jax.experimental import pallas as pl
from jax.experimental.pallas import tpu as pltpu
from jax.experimental.pallas import tpu_sc as plsc

N = 10000
E = 320000
D = 128
ACCR = 10120
DEGP = 10240

NC = 2
NS = 16
NW = NC * NS
EPW = E // NW
K = 80
NCHUNK = EPW // K
NBLK = 5
CB = NCHUNK // NBLK
NPAIRB = (CB - 1) // 2
RPT = 632


def _sc_aggregate(src_hbm, dst_hbm, nodes_hbm, ef_hbm, s_out, deg_out,
                  sidx_blk, didx_blk, rows_na, rows_ea, rows_nb, rows_eb,
                  ones_v, acc_sh, deg_sh,
                  gs_a, es_a, ss_a, gs_b, es_b, ss_b, dg_s):
    c = lax.axis_index("c")
    s = lax.axis_index("s")
    wid = s * NC + c

    zero16 = jnp.zeros((16,), jnp.float32)

    def zero_rows(r, _):
        def zero_cols(j, _):
            rows_na[r, pl.ds(j * 16, 16)] = zero16
            return 0
        return lax.fori_loop(0, D // 16, zero_cols, 0)

    lax.fori_loop(0, K, zero_rows, 0)

    def zero_ones(i, _):
        ones_v[pl.ds(i * 16, 16)] = zero16
        return 0
    lax.fori_loop(0, K // 16, zero_ones, 0)

    rbase = s * RPT
    dbase = s * (DEGP // NS)
    for t in range(640 // K):
        pltpu.sync_copy(rows_na.at[pl.ds(0, K)],
                        acc_sh.at[pl.ds(rbase + t * K, K)])
        pltpu.sync_copy(ones_v.at[pl.ds(0, K)],
                        deg_sh.at[pl.ds(dbase + t * K, K)])

    one16 = jnp.ones((16,), jnp.float32)

    def fill_ones(i, _):
        ones_v[pl.ds(i * 16, 16)] = one16
        return 0
    lax.fori_loop(0, K // 16, fill_ones, 0)

    plsc.subcore_barrier()

    def issue_loads(blk, j, rows_n, rows_e, gsem, esem):
        gd = pltpu.async_copy(nodes_hbm.at[sidx_blk.at[j]], rows_n, gsem)
        off = ((wid * NCHUNK + blk * CB + j) * K)
        ed = pltpu.async_copy(ef_hbm.at[pl.ds(off, K)], rows_e, esem)
        return gd, ed

    def issue_scatters(j, rows_n, rows_e, ssem):
        pltpu.async_copy(rows_n, acc_sh.at[didx_blk.at[j]], ssem, add=True)
        pltpu.async_copy(rows_e, acc_sh.at[didx_blk.at[j]], ssem, add=True)
        pltpu.async_copy(ones_v, deg_sh.at[didx_blk.at[j]], dg_s, add=True)

    def wait_scatters(j, rows_n, rows_e, ssem):
        pltpu.make_async_copy(rows_n, acc_sh.at[didx_blk.at[j]], ssem).wait()
        pltpu.make_async_copy(rows_e, acc_sh.at[didx_blk.at[j]], ssem).wait()

    def block(blk, _):
        pltpu.sync_copy(src_hbm.at[wid, blk], sidx_blk)
        pltpu.sync_copy(dst_hbm.at[wid, blk], didx_blk)

        def pair(i, _):
            ja = 2 * i
            jb = 2 * i + 1
            gd_a, ed_a = issue_loads(blk, ja, rows_na, rows_ea, gs_a, es_a)

            @pl.when(i > 0)
            def _():
                wait_scatters(jb - 2, rows_nb, rows_eb, ss_b)

            gd_b, ed_b = issue_loads(blk, jb, rows_nb, rows_eb, gs_b, es_b)
            gd_a.wait()
            ed_a.wait()
            issue_scatters(ja, rows_na, rows_ea, ss_a)
            gd_b.wait()
            ed_b.wait()
            issue_scatters(jb, rows_nb, rows_eb, ss_b)
            wait_scatters(ja, rows_na, rows_ea, ss_a)
            return 0

        lax.fori_loop(0, NPAIRB, pair, 0)

        jl = CB - 1
        gd_a, ed_a = issue_loads(blk, jl, rows_na, rows_ea, gs_a, es_a)
        wait_scatters(jl - 2, rows_nb, rows_eb, ss_b)
        gd_a.wait()
        ed_a.wait()
        issue_scatters(jl, rows_na, rows_ea, ss_a)
        wait_scatters(jl, rows_na, rows_ea, ss_a)

        def drain_deg(j, _):
            pltpu.make_async_copy(ones_v, deg_sh.at[didx_blk.at[j]],
                                  dg_s).wait()
            return 0
        lax.fori_loop(0, CB, drain_deg, 0)
        return 0

    lax.fori_loop(0, NBLK, block, 0)

    plsc.subcore_barrier()

    pltpu.sync_copy(deg_sh.at[pl.ds(dbase, DEGP // NS)],
                    deg_out.at[pl.ds(c * DEGP + dbase, DEGP // NS)])

    @pl.when(s < NS - 1)
    def _():
        pltpu.sync_copy(acc_sh.at[pl.ds(rbase, RPT)],
                        s_out.at[c, pl.ds(rbase, RPT)])

    @pl.when(s == NS - 1)
    def _():
        pltpu.sync_copy(acc_sh.at[pl.ds((NS - 1) * RPT, N - (NS - 1) * RPT)],
                        s_out.at[c, pl.ds((NS - 1) * RPT, N - (NS - 1) * RPT)])


_sc_call = functools.partial(
    pl.kernel,
    out_type=[
        jax.ShapeDtypeStruct((NC, N, D), jnp.float32),
        jax.ShapeDtypeStruct((NC * DEGP,), jnp.float32),
    ],
    mesh=plsc.VectorSubcoreMesh(core_axis_name="c", subcore_axis_name="s"),
    scratch_types=[
        pltpu.VMEM((CB, K), jnp.int32),
        pltpu.VMEM((CB, K), jnp.int32),
        pltpu.VMEM((K, D), jnp.float32),
        pltpu.VMEM((K, D), jnp.float32),
        pltpu.VMEM((K, D), jnp.float32),
        pltpu.VMEM((K, D), jnp.float32),
        pltpu.VMEM((K,), jnp.float32),
        pltpu.VMEM_SHARED((ACCR, D), jnp.float32),
        pltpu.VMEM_SHARED((DEGP,), jnp.float32),
        pltpu.SemaphoreType.DMA,
        pltpu.SemaphoreType.DMA,
        pltpu.SemaphoreType.DMA,
        pltpu.SemaphoreType.DMA,
        pltpu.SemaphoreType.DMA,
        pltpu.SemaphoreType.DMA,
        pltpu.SemaphoreType.DMA,
    ],
)(_sc_aggregate)


def _bn_relu(h, g, beta):
    mean = jnp.mean(h, axis=0, keepdims=True)
    var = jnp.mean(h * h, axis=0, keepdims=True) - mean * mean
    return jnp.maximum(g * (h - mean) * lax.rsqrt(var + 1e-5) + beta, 0.0)


def _tc_mlp(x_ref, sp_ref, dp_ref, eps_ref, w1_ref, b1_ref, g1_ref, bt1_ref,
            w2_ref, b2_ref, g2_ref, bt2_ref, o_ref):
    s = sp_ref[0] + sp_ref[1]
    deg = dp_ref[:N] + dp_ref[DEGP:DEGP + N]
    h = ((1.0 + eps_ref[0, 0]) * x_ref[...]
         + s / jnp.maximum(deg[:, None], 1.0))

    h = jnp.dot(h.astype(jnp.bfloat16), w1_ref[...].astype(jnp.bfloat16),
                preferred_element_type=jnp.float32) + b1_ref[...]
    h = _bn_relu(h, g1_ref[...], bt1_ref[...])
    h = jnp.dot(h.astype(jnp.bfloat16), w2_ref[...].astype(jnp.bfloat16),
                preferred_element_type=jnp.float32) + b2_ref[...]
    o_ref[...] = _bn_relu(h, g2_ref[...], bt2_ref[...])


def kernel(node_feats, edge_index, edge_feats, eps, W1, b1, g1, beta1,
           W2, b2, g2, beta2):
    src = edge_index[0].reshape(NW, NBLK, CB, K)
    dst = edge_index[1].reshape(NW, NBLK, CB, K)

    s_part, deg_part = _sc_call(src, dst, node_feats, edge_feats)

    out = pl.pallas_call(
        _tc_mlp,
        out_shape=jax.ShapeDtypeStruct((N, D), jnp.float32),
    )(
        node_feats, s_part, deg_part,
        eps.reshape(1, 1),
        W1, b1.reshape(1, -1), g1.reshape(1, -1), beta1.reshape(1, -1),
        W2, b2.reshape(1, -1), g2.reshape(1, -1), beta2.reshape(1, -1),
    )
    return out
